# baseline (speedup 1.0000x reference)
import functools

import jax
import jax.numpy as jnp
from jax import lax
from jax.experimental import pallas as pl
from jax.experimental.pallas import tpu as pltpu
from jax.experimental.pallas import tpu_sc as plsc

NC = 2
NS = 16
NW = NC * NS
CHUNK = 128
BLK = 8192
L16 = 16


def _proj_table_kernel(vt_ref, w_ref, b_ref, out_ref):
    y = lax.dot_general(
        vt_ref[...], w_ref[...], (((0,), (1,)), ((), ())),
        preferred_element_type=jnp.float32,
    ) + b_ref[...]
    yh = y.astype(jnp.bfloat16)
    h = yh.shape[0] // 2
    out_ref[...] = jnp.concatenate([yh[:h, :], yh[h:, :]], axis=1)


def _tc_project_table(vectors, W, b):
    vocab, vec_dim = vectors.shape
    size = W.shape[0]
    vt = vectors.T
    n_blocks = pl.cdiv(vocab, BLK)
    return pl.pallas_call(
        _proj_table_kernel,
        grid=(n_blocks,),
        in_specs=[
            pl.BlockSpec((vec_dim, BLK), lambda i: (0, i)),
            pl.BlockSpec((size, vec_dim), lambda i: (0, 0)),
            pl.BlockSpec((1, size), lambda i: (0, 0)),
        ],
        out_specs=pl.BlockSpec((BLK // 2, size * 2), lambda i: (i, 0)),
        out_shape=jax.ShapeDtypeStruct((n_blocks * BLK // 2, size * 2),
                                       jnp.bfloat16),
    )(vt, W, b.reshape(1, size))


def _gather_kernel(n_chunks, tokens_hbm, table_hbm, out_hbm,
                   idx_v, rows_a, rows_b, sem_a, sem_b):
    wid = lax.axis_index("s") * NC + lax.axis_index("c")
    pltpu.sync_copy(tokens_hbm.at[wid], idx_v)

    def xform(j, _):
        c = j // (CHUNK // L16)
        o = (j % (CHUNK // L16)) * L16
        t = idx_v[c, pl.ds(o, L16)]
        g = jax.lax.shift_right_logical(t, 13)
        hi = jax.lax.shift_right_logical(t, 12) & 1
        im = t & 4095
        slot = jax.lax.shift_left(jax.lax.shift_left(g, 12) | im, 1) | hi
        idx_v[c, pl.ds(o, L16)] = slot
        return 0

    lax.fori_loop(0, n_chunks * (CHUNK // L16), xform, 0)

    def fire(c, rows, sem):
        return pltpu.async_copy(table_hbm.at[idx_v.at[c]], rows, sem)

    def store(c, rows):
        pltpu.sync_copy(rows, out_hbm.at[wid, c])

    fire(0, rows_a, sem_a)

    def body(i, _):
        c = i * 2
        fire(c + 1, rows_b, sem_b)
        pltpu.make_async_copy(table_hbm.at[idx_v.at[c]], rows_a, sem_a).wait()
        store(c, rows_a)

        @pl.when(c + 2 < n_chunks)
        def _():
            fire(c + 2, rows_a, sem_a)

        pltpu.make_async_copy(table_hbm.at[idx_v.at[c + 1]], rows_b, sem_b).wait()
        store(c + 1, rows_b)
        return 0

    lax.fori_loop(0, n_chunks // 2, body, 0)


def _sc_gather(tokens_flat, table):
    n_rows = tokens_flat.shape[0]
    vec_dim = table.shape[1]
    per_w = n_rows // NW
    n_chunks = per_w // CHUNK
    tokens3 = tokens_flat.reshape(NW, n_chunks, CHUNK)
    mesh = plsc.VectorSubcoreMesh(core_axis_name="c", subcore_axis_name="s")
    kern = pl.kernel(
        functools.partial(_gather_kernel, n_chunks),
        out_type=jax.ShapeDtypeStruct((NW, n_chunks, CHUNK, vec_dim),
                                      jnp.bfloat16),
        mesh=mesh,
        scratch_types=[
            pltpu.VMEM((n_chunks, CHUNK), jnp.int32),
            pltpu.VMEM((CHUNK, vec_dim), jnp.bfloat16),
            pltpu.VMEM((CHUNK, vec_dim), jnp.bfloat16),
            pltpu.SemaphoreType.DMA,
            pltpu.SemaphoreType.DMA,
        ],
        compiler_params=pltpu.CompilerParams(use_tc_tiling_on_sc=False),
    )
    out4 = kern(tokens3, table)
    return out4.reshape(n_rows, vec_dim)


def kernel(tokens, vectors, W, b):
    bsz, l = tokens.shape
    size = W.shape[0]
    tokens_flat = tokens.reshape(-1)
    table_pairs = _tc_project_table(vectors, W, b)
    table = table_pairs.reshape(table_pairs.shape[0] * 2, size)
    proj = _sc_gather(tokens_flat, table)
    return proj.astype(jnp.float32).reshape(bsz, l, -1)


# speedup vs baseline: 2.0465x; 2.0465x over previous
import functools

import jax
import jax.numpy as jnp
from jax import lax
from jax.experimental import pallas as pl
from jax.experimental.pallas import tpu as pltpu
from jax.experimental.pallas import tpu_sc as plsc

NC = 2
NS = 16
NW = NC * NS
CHUNK = 128
BLK = 8192
L16 = 16


def _proj_table_kernel(vt_ref, w_ref, b_ref, out_ref):
    y = lax.dot_general(
        vt_ref[...], w_ref[...], (((0,), (1,)), ((), ())),
        preferred_element_type=jnp.float32,
    ) + b_ref[...]
    h = y.shape[0] // 2
    out_ref[...] = jnp.concatenate([y[:h, :], y[h:, :]], axis=1)


def _tc_project_table(vectors, W, b):
    vocab, vec_dim = vectors.shape
    size = W.shape[0]
    vt = vectors.T
    n_blocks = pl.cdiv(vocab, BLK)
    return pl.pallas_call(
        _proj_table_kernel,
        grid=(n_blocks,),
        in_specs=[
            pl.BlockSpec((vec_dim, BLK), lambda i: (0, i)),
            pl.BlockSpec((size, vec_dim), lambda i: (0, 0)),
            pl.BlockSpec((1, size), lambda i: (0, 0)),
        ],
        out_specs=pl.BlockSpec((BLK // 2, size * 2), lambda i: (i, 0)),
        out_shape=jax.ShapeDtypeStruct((n_blocks * BLK // 2, size * 2),
                                       jnp.float32),
    )(vt, W, b.reshape(1, size))


def _gather_kernel(n_chunks, tokens_hbm, table_hbm, out_hbm,
                   idx_v, rows_a, rows_b, sem_a, sem_b, sem_sa, sem_sb):
    wid = lax.axis_index("s") * NC + lax.axis_index("c")
    pltpu.sync_copy(tokens_hbm.at[wid], idx_v)

    def xform(j, _):
        c = j // (CHUNK // L16)
        o = (j % (CHUNK // L16)) * L16
        t = idx_v[c, pl.ds(o, L16)]
        g = jax.lax.shift_right_logical(t, 13)
        hi = jax.lax.shift_right_logical(t, 12) & 1
        im = t & 4095
        slot = jax.lax.shift_left(jax.lax.shift_left(g, 12) | im, 1) | hi
        idx_v[c, pl.ds(o, L16)] = slot
        return 0

    lax.fori_loop(0, n_chunks * (CHUNK // L16), xform, 0)

    def fire(c, rows, sem):
        return pltpu.async_copy(table_hbm.at[idx_v.at[c]], rows, sem)

    def store(c, rows, sem):
        return pltpu.async_copy(rows, out_hbm.at[wid, c], sem)

    def wait_g(c, rows, sem):
        pltpu.make_async_copy(table_hbm.at[idx_v.at[c]], rows, sem).wait()

    def wait_s(c, rows, sem):
        pltpu.make_async_copy(rows, out_hbm.at[wid, c], sem).wait()

    fire(0, rows_a, sem_a)

    def body(i, _):
        c = i * 2

        @pl.when(c > 0)
        def _():
            wait_s(c - 1, rows_b, sem_sb)

        fire(c + 1, rows_b, sem_b)
        wait_g(c, rows_a, sem_a)
        store(c, rows_a, sem_sa)

        wait_g(c + 1, rows_b, sem_b)
        wait_s(c, rows_a, sem_sa)

        @pl.when(c + 2 < n_chunks)
        def _():
            fire(c + 2, rows_a, sem_a)

        store(c + 1, rows_b, sem_sb)
        return 0

    lax.fori_loop(0, n_chunks // 2, body, 0)
    wait_s(n_chunks - 1, rows_b, sem_sb)


def _sc_gather(tokens_flat, table):
    n_rows = tokens_flat.shape[0]
    vec_dim = table.shape[1]
    per_w = n_rows // NW
    n_chunks = per_w // CHUNK
    tokens3 = tokens_flat.reshape(NW, n_chunks, CHUNK)
    mesh = plsc.VectorSubcoreMesh(core_axis_name="c", subcore_axis_name="s")
    kern = pl.kernel(
        functools.partial(_gather_kernel, n_chunks),
        out_type=jax.ShapeDtypeStruct((NW, n_chunks, CHUNK, vec_dim),
                                      jnp.float32),
        mesh=mesh,
        scratch_types=[
            pltpu.VMEM((n_chunks, CHUNK), jnp.int32),
            pltpu.VMEM((CHUNK, vec_dim), jnp.float32),
            pltpu.VMEM((CHUNK, vec_dim), jnp.float32),
            pltpu.SemaphoreType.DMA,
            pltpu.SemaphoreType.DMA,
            pltpu.SemaphoreType.DMA,
            pltpu.SemaphoreType.DMA,
        ],
        compiler_params=pltpu.CompilerParams(use_tc_tiling_on_sc=False),
    )
    out4 = kern(tokens3, table)
    return out4.reshape(n_rows, vec_dim)


def kernel(tokens, vectors, W, b):
    bsz, l = tokens.shape
    size = W.shape[0]
    tokens_flat = tokens.reshape(-1)
    table_pairs = _tc_project_table(vectors, W, b)
    table = table_pairs.reshape(table_pairs.shape[0] * 2, size)
    proj = _sc_gather(tokens_flat, table)
    return proj.reshape(bsz, l, -1)


# speedup vs baseline: 2.2057x; 1.0778x over previous
import functools

import jax
import jax.numpy as jnp
from jax import lax
from jax.experimental import pallas as pl
from jax.experimental.pallas import tpu as pltpu
from jax.experimental.pallas import tpu_sc as plsc

NC = 2
NS = 16
NW = NC * NS
CHUNK = 128
BLK = 16384
LB = BLK.bit_length() - 1
HMASK = BLK // 2 - 1
L16 = 16


def _proj_table_kernel(vt_ref, w_ref, b_ref, out_ref):
    y = lax.dot_general(
        vt_ref[...], w_ref[...], (((0,), (1,)), ((), ())),
        preferred_element_type=jnp.float32,
    ) + b_ref[...]
    h = y.shape[0] // 2
    out_ref[...] = jnp.concatenate([y[:h, :], y[h:, :]], axis=1)


def _tc_project_table(vectors, W, b):
    vocab, vec_dim = vectors.shape
    size = W.shape[0]
    vt = vectors.T
    n_blocks = pl.cdiv(vocab, BLK)
    return pl.pallas_call(
        _proj_table_kernel,
        grid=(n_blocks,),
        in_specs=[
            pl.BlockSpec((vec_dim, BLK), lambda i: (0, i)),
            pl.BlockSpec((size, vec_dim), lambda i: (0, 0)),
            pl.BlockSpec((1, size), lambda i: (0, 0)),
        ],
        out_specs=pl.BlockSpec((BLK // 2, size * 2), lambda i: (i, 0)),
        out_shape=jax.ShapeDtypeStruct((n_blocks * BLK // 2, size * 2),
                                       jnp.float32),
    )(vt, W, b.reshape(1, size))


def _gather_kernel(n_chunks, tokens_hbm, table_hbm, out_hbm,
                   idx_v, rows_a, rows_b, sem_a, sem_b, sem_sa, sem_sb):
    wid = lax.axis_index("s") * NC + lax.axis_index("c")
    pltpu.sync_copy(tokens_hbm.at[wid], idx_v)

    def xform(j, _):
        c = j // (CHUNK // L16)
        o = (j % (CHUNK // L16)) * L16
        t = idx_v[c, pl.ds(o, L16)]
        g = jax.lax.shift_right_logical(t, LB)
        hi = jax.lax.shift_right_logical(t, LB - 1) & 1
        im = t & HMASK
        slot = jax.lax.shift_left(jax.lax.shift_left(g, LB - 1) | im, 1) | hi
        idx_v[c, pl.ds(o, L16)] = slot
        return 0

    lax.fori_loop(0, n_chunks * (CHUNK // L16), xform, 0)

    def fire(c, rows, sem):
        return pltpu.async_copy(table_hbm.at[idx_v.at[c]], rows, sem)

    def store(c, rows, sem):
        return pltpu.async_copy(rows, out_hbm.at[wid, c], sem)

    def wait_g(c, rows, sem):
        pltpu.make_async_copy(table_hbm.at[idx_v.at[c]], rows, sem).wait()

    def wait_s(c, rows, sem):
        pltpu.make_async_copy(rows, out_hbm.at[wid, c], sem).wait()

    fire(0, rows_a, sem_a)

    def body(i, _):
        c = i * 2

        @pl.when(c > 0)
        def _():
            wait_s(c - 1, rows_b, sem_sb)

        fire(c + 1, rows_b, sem_b)
        wait_g(c, rows_a, sem_a)
        store(c, rows_a, sem_sa)

        wait_g(c + 1, rows_b, sem_b)
        wait_s(c, rows_a, sem_sa)

        @pl.when(c + 2 < n_chunks)
        def _():
            fire(c + 2, rows_a, sem_a)

        store(c + 1, rows_b, sem_sb)
        return 0

    lax.fori_loop(0, n_chunks // 2, body, 0)
    wait_s(n_chunks - 1, rows_b, sem_sb)


def _sc_gather(tokens_flat, table):
    n_rows = tokens_flat.shape[0]
    vec_dim = table.shape[1]
    per_w = n_rows // NW
    n_chunks = per_w // CHUNK
    tokens3 = tokens_flat.reshape(NW, n_chunks, CHUNK)
    mesh = plsc.VectorSubcoreMesh(core_axis_name="c", subcore_axis_name="s")
    kern = pl.kernel(
        functools.partial(_gather_kernel, n_chunks),
        out_type=jax.ShapeDtypeStruct((NW, n_chunks, CHUNK, vec_dim),
                                      jnp.float32),
        mesh=mesh,
        scratch_types=[
            pltpu.VMEM((n_chunks, CHUNK), jnp.int32),
            pltpu.VMEM((CHUNK, vec_dim), jnp.float32),
            pltpu.VMEM((CHUNK, vec_dim), jnp.float32),
            pltpu.SemaphoreType.DMA,
            pltpu.SemaphoreType.DMA,
            pltpu.SemaphoreType.DMA,
            pltpu.SemaphoreType.DMA,
        ],
        compiler_params=pltpu.CompilerParams(use_tc_tiling_on_sc=False),
    )
    out4 = kern(tokens3, table)
    return out4.reshape(n_rows, vec_dim)


def kernel(tokens, vectors, W, b):
    bsz, l = tokens.shape
    size = W.shape[0]
    tokens_flat = tokens.reshape(-1)
    table_pairs = _tc_project_table(vectors, W, b)
    table = table_pairs.reshape(table_pairs.shape[0] * 2, size)
    proj = _sc_gather(tokens_flat, table)
    return proj.reshape(bsz, l, -1)


# speedup vs baseline: 2.2858x; 1.0363x over previous
import functools

import jax
import jax.numpy as jnp
from jax import lax
from jax.experimental import pallas as pl
from jax.experimental.pallas import tpu as pltpu
from jax.experimental.pallas import tpu_sc as plsc

NC = 2
NS = 16
NW = NC * NS
CHUNK = 128
BLK = 32768
LB = BLK.bit_length() - 1
HMASK = BLK // 2 - 1
L16 = 16


def _proj_table_kernel(vt_ref, w_ref, b_ref, out_ref):
    y = lax.dot_general(
        vt_ref[...], w_ref[...], (((0,), (1,)), ((), ())),
        preferred_element_type=jnp.float32,
    ) + b_ref[...]
    h = y.shape[0] // 2
    out_ref[...] = jnp.concatenate([y[:h, :], y[h:, :]], axis=1)


def _tc_project_table(vectors, W, b):
    vocab, vec_dim = vectors.shape
    size = W.shape[0]
    vt = vectors.T
    n_blocks = pl.cdiv(vocab, BLK)
    return pl.pallas_call(
        _proj_table_kernel,
        grid=(n_blocks,),
        in_specs=[
            pl.BlockSpec((vec_dim, BLK), lambda i: (0, i)),
            pl.BlockSpec((size, vec_dim), lambda i: (0, 0)),
            pl.BlockSpec((1, size), lambda i: (0, 0)),
        ],
        out_specs=pl.BlockSpec((BLK // 2, size * 2), lambda i: (i, 0)),
        out_shape=jax.ShapeDtypeStruct((n_blocks * BLK // 2, size * 2),
                                       jnp.float32),
    )(vt, W, b.reshape(1, size))


def _gather_kernel(n_chunks, tokens_hbm, table_hbm, out_hbm,
                   idx_v, rows_a, rows_b, sem_a, sem_b, sem_sa, sem_sb):
    wid = lax.axis_index("s") * NC + lax.axis_index("c")
    pltpu.sync_copy(tokens_hbm.at[wid], idx_v)

    def xform(j, _):
        c = j // (CHUNK // L16)
        o = (j % (CHUNK // L16)) * L16
        t = idx_v[c, pl.ds(o, L16)]
        g = jax.lax.shift_right_logical(t, LB)
        hi = jax.lax.shift_right_logical(t, LB - 1) & 1
        im = t & HMASK
        slot = jax.lax.shift_left(jax.lax.shift_left(g, LB - 1) | im, 1) | hi
        idx_v[c, pl.ds(o, L16)] = slot
        return 0

    lax.fori_loop(0, n_chunks * (CHUNK // L16), xform, 0)

    def fire(c, rows, sem):
        return pltpu.async_copy(table_hbm.at[idx_v.at[c]], rows, sem)

    def store(c, rows, sem):
        return pltpu.async_copy(rows, out_hbm.at[wid, c], sem)

    def wait_g(c, rows, sem):
        pltpu.make_async_copy(table_hbm.at[idx_v.at[c]], rows, sem).wait()

    def wait_s(c, rows, sem):
        pltpu.make_async_copy(rows, out_hbm.at[wid, c], sem).wait()

    fire(0, rows_a, sem_a)

    def body(i, _):
        c = i * 2

        @pl.when(c > 0)
        def _():
            wait_s(c - 1, rows_b, sem_sb)

        fire(c + 1, rows_b, sem_b)
        wait_g(c, rows_a, sem_a)
        store(c, rows_a, sem_sa)

        wait_g(c + 1, rows_b, sem_b)
        wait_s(c, rows_a, sem_sa)

        @pl.when(c + 2 < n_chunks)
        def _():
            fire(c + 2, rows_a, sem_a)

        store(c + 1, rows_b, sem_sb)
        return 0

    lax.fori_loop(0, n_chunks // 2, body, 0)
    wait_s(n_chunks - 1, rows_b, sem_sb)


def _sc_gather(tokens_flat, table):
    n_rows = tokens_flat.shape[0]
    vec_dim = table.shape[1]
    per_w = n_rows // NW
    n_chunks = per_w // CHUNK
    tokens3 = tokens_flat.reshape(NW, n_chunks, CHUNK)
    mesh = plsc.VectorSubcoreMesh(core_axis_name="c", subcore_axis_name="s")
    kern = pl.kernel(
        functools.partial(_gather_kernel, n_chunks),
        out_type=jax.ShapeDtypeStruct((NW, n_chunks, CHUNK, vec_dim),
                                      jnp.float32),
        mesh=mesh,
        scratch_types=[
            pltpu.VMEM((n_chunks, CHUNK), jnp.int32),
            pltpu.VMEM((CHUNK, vec_dim), jnp.float32),
            pltpu.VMEM((CHUNK, vec_dim), jnp.float32),
            pltpu.SemaphoreType.DMA,
            pltpu.SemaphoreType.DMA,
            pltpu.SemaphoreType.DMA,
            pltpu.SemaphoreType.DMA,
        ],
        compiler_params=pltpu.CompilerParams(use_tc_tiling_on_sc=False),
    )
    out4 = kern(tokens3, table)
    return out4.reshape(n_rows, vec_dim)


def kernel(tokens, vectors, W, b):
    bsz, l = tokens.shape
    size = W.shape[0]
    tokens_flat = tokens.reshape(-1)
    table_pairs = _tc_project_table(vectors, W, b)
    table = table_pairs.reshape(table_pairs.shape[0] * 2, size)
    proj = _sc_gather(tokens_flat, table)
    return proj.reshape(bsz, l, -1)
